# baseline (speedup 1.0000x reference)
import functools

import jax
import jax.numpy as jnp
from jax import lax
from jax.experimental import pallas as pl
from jax.experimental.pallas import tpu as pltpu
from jax.experimental.pallas import tpu_sc as plsc

LANES = 16
SUB = 8
LN = 128
TILE = SUB * LN


def _pos_embed_body(nb, h, w, d, f, nc,
                    colf_hbm, rowf_hbm, depf_hbm, out_hbm,
                    colv, rowv, depv, slab0, slab1, sem0, sem1):
    dt_n = d // SUB
    ct_n = (3 * f) // LN
    slab_len = dt_n * ct_n * TILE
    jn = f // LANES

    hh = lax.axis_index("s") * nc + lax.axis_index("c")

    pltpu.sync_copy(colf_hbm.at[pl.ds(0, w * f)], colv)
    pltpu.sync_copy(depf_hbm.at[pl.ds(0, d * f)], depv)
    pltpu.sync_copy(rowf_hbm.at[pl.ds(hh * f, f)], rowv)

    row_regs = [rowv[pl.ds(j * LANES, LANES)] for j in range(jn)]
    for buf in (slab0, slab1):
        for dt in range(dt_n):
            b1 = (dt * ct_n + 1) * TILE
            b2 = (dt * ct_n + 2) * TILE
            for dr in range(SUB):
                for j in range(jn):
                    o = dr * LN + j * LANES
                    buf[pl.ds(b1 + o, LANES)] = row_regs[j]
                    buf[pl.ds(b2 + o, LANES)] = depv[pl.ds(dt * TILE + o,
                                                           LANES)]

    def patch_col(buf, wq):
        col_regs = [colv[pl.ds(wq * f + j * LANES, LANES)] for j in range(jn)]
        for dt in range(dt_n):
            b0 = dt * ct_n * TILE
            for dr in range(SUB):
                for j in range(jn):
                    buf[pl.ds(b0 + dr * LN + j * LANES, LANES)] = col_regs[j]

    def fire(buf, wq, sem):
        for b in range(nb):
            off = ((b * h + hh) * w + wq) * slab_len
            pltpu.async_copy(buf, out_hbm.at[pl.ds(off, slab_len)], sem)

    def drain(buf, sem):
        for _ in range(nb):
            pltpu.make_async_copy(
                buf, out_hbm.at[pl.ds(0, slab_len)], sem).wait()

    patch_col(slab0, 0)
    fire(slab0, 0, sem0)
    patch_col(slab1, 1)
    fire(slab1, 1, sem1)

    def pair(k, carry):
        wq = 2 * k
        drain(slab0, sem0)
        patch_col(slab0, wq)
        fire(slab0, wq, sem0)
        drain(slab1, sem1)
        patch_col(slab1, wq + 1)
        fire(slab1, wq + 1, sem1)
        return carry

    lax.fori_loop(1, w // 2, pair, 0)

    drain(slab0, sem0)
    drain(slab1, sem1)


def kernel(tensor_list, row_embed_weight, col_embed_weight, depth_embed_weight):
    x = tensor_list
    h, w, d = x.shape[-3], x.shape[-2], x.shape[-1]
    nb = x.shape[0]
    f = row_embed_weight.shape[-1]
    n_chan = 3 * f

    info = plsc.get_sparse_core_info()
    nc, ns = info.num_cores, info.num_subcores
    assert nc * ns == h, "one vector subcore per h plane"

    colf = col_embed_weight.reshape(-1)
    rowf = row_embed_weight.reshape(-1)
    depf = depth_embed_weight.reshape(-1)

    dt_n = d // SUB
    ct_n = n_chan // LN
    total = nb * h * w * dt_n * ct_n * TILE

    run = pl.kernel(
        functools.partial(_pos_embed_body, nb, h, w, d, f, nc),
        mesh=plsc.VectorSubcoreMesh(core_axis_name="c", subcore_axis_name="s"),
        out_type=jax.ShapeDtypeStruct((total,), jnp.float32),
        scratch_types=[
            pltpu.VMEM((w * f,), jnp.float32),
            pltpu.VMEM((f,), jnp.float32),
            pltpu.VMEM((d * f,), jnp.float32),
            pltpu.VMEM((dt_n * ct_n * TILE,), jnp.float32),
            pltpu.VMEM((dt_n * ct_n * TILE,), jnp.float32),
            pltpu.SemaphoreType.DMA,
            pltpu.SemaphoreType.DMA,
        ],
    )
    out = run(colf, rowf, depf)
    out7 = out.reshape(nb, h, w, dt_n, ct_n, SUB, LN)
    out5 = out7.transpose(0, 4, 6, 1, 2, 3, 5).reshape(nb, n_chan, h, w, d)
    return out5
